# baseline (speedup 1.0000x reference)
import functools

import jax
import jax.numpy as jnp
from jax import lax
from jax.experimental import pallas as pl
from jax.experimental.pallas import tpu as pltpu
from jax.experimental.pallas import tpu_sc as plsc

N = 100000
DIM = 128
K = 64
NC = 2
NS = 16
L = 16

RPT = N // NS
CHUNK = 250
NCHUNKS = RPT // CHUNK
NBUF = 4
GRP = 125
GPC = CHUNK // GRP
YROWS = N // GRP
YPT = RPT // GRP
KPT = K // NS
CW = DIM // NC


def _seg_mean_body(x_hbm, y_hbm, out_hbm, xb0, xb1, xb2, xb3, ybuf,
                   ones, wbuf, cbuf, acc, cacc, sf0, sf1, sf2, sf3,
                   ss0, ss1, ss2, ss3, scnt):
    c = lax.axis_index("c")
    s = lax.axis_index("s")

    zero16 = jnp.zeros((L,), jnp.float32)
    one16 = jnp.ones((L,), jnp.float32)
    for r in range(KPT):
        for j in range(CW // L):
            wbuf[r, pl.ds(L * j, L)] = zero16
        cbuf[r, :] = zero16
    for r in range(GRP):
        ones[r, :] = one16

    pltpu.sync_copy(wbuf, acc.at[pl.ds(s * KPT, KPT)])
    pltpu.sync_copy(cbuf, cacc.at[pl.ds(s * KPT, KPT)])
    pltpu.sync_copy(y_hbm.at[pl.ds(s * YPT, YPT)], ybuf)
    plsc.subcore_barrier()

    xb = (xb0, xb1, xb2, xb3)
    sf = (sf0, sf1, sf2, sf3)
    ss = (ss0, ss1, ss2, ss3)

    def fill(j):
        row0 = s * RPT + j * CHUNK
        return pltpu.async_copy(
            x_hbm.at[pl.ds(row0, CHUNK), pl.ds(c * CW, CW)],
            xb[j % NBUF], sf[j % NBUF])

    fills = {}
    for j in range(NBUF):
        fills[j] = fill(j)
    cnt_handles = []
    for j in range(NCHUNKS):
        b = j % NBUF
        fills[j].wait()
        xscat = []
        for g in range(GPC):
            idx = ybuf.at[j * GPC + g]
            xscat.append(pltpu.async_copy(xb[b].at[pl.ds(g * GRP, GRP)],
                                          acc.at[idx], ss[b], add=True))
            cnt_handles.append(pltpu.async_copy(ones, cacc.at[idx],
                                                scnt, add=True))
        for h in xscat:
            h.wait()
        if j + NBUF < NCHUNKS:
            fills[j + NBUF] = fill(j + NBUF)
    for h in cnt_handles:
        h.wait()

    plsc.subcore_barrier()

    pltpu.sync_copy(acc.at[pl.ds(s * KPT, KPT)], wbuf)
    pltpu.sync_copy(cacc.at[pl.ds(s * KPT, KPT)], cbuf)
    for r in range(KPT):
        cnt = jnp.maximum(cbuf[r, :], 1.0)
        for j in range(CW // L):
            wbuf[r, pl.ds(L * j, L)] = wbuf[r, pl.ds(L * j, L)] / cnt
    pltpu.sync_copy(wbuf, out_hbm.at[pl.ds(s * KPT, KPT),
                                     pl.ds(c * CW, CW)])


@jax.jit
def _seg_mean(x, y2):
    return pl.kernel(
        _seg_mean_body,
        out_type=jax.ShapeDtypeStruct((K, DIM), jnp.float32),
        mesh=plsc.VectorSubcoreMesh(core_axis_name="c",
                                    subcore_axis_name="s"),
        scratch_types=[
            pltpu.VMEM((CHUNK, CW), jnp.float32),
            pltpu.VMEM((CHUNK, CW), jnp.float32),
            pltpu.VMEM((CHUNK, CW), jnp.float32),
            pltpu.VMEM((CHUNK, CW), jnp.float32),
            pltpu.VMEM((YPT, GRP), jnp.int32),
            pltpu.VMEM((GRP, L), jnp.float32),
            pltpu.VMEM((KPT, CW), jnp.float32),
            pltpu.VMEM((KPT, L), jnp.float32),
            pltpu.VMEM_SHARED((K, CW), jnp.float32),
            pltpu.VMEM_SHARED((K, L), jnp.float32),
            pltpu.SemaphoreType.DMA,
            pltpu.SemaphoreType.DMA,
            pltpu.SemaphoreType.DMA,
            pltpu.SemaphoreType.DMA,
            pltpu.SemaphoreType.DMA,
            pltpu.SemaphoreType.DMA,
            pltpu.SemaphoreType.DMA,
            pltpu.SemaphoreType.DMA,
            pltpu.SemaphoreType.DMA,
        ],
        compiler_params=pltpu.CompilerParams(use_tc_tiling_on_sc=False),
    )(x, y2)


def kernel(x, y):
    y2 = y.astype(jnp.int32).reshape(YROWS, GRP)
    weights = _seg_mean(x, y2)
    moving_avg = jnp.zeros((K, DIM), dtype=jnp.float32)
    relevances = jnp.ones((K, DIM), dtype=jnp.float32)
    neighbors = jnp.zeros((K, K), dtype=jnp.uint8)
    wins = jnp.zeros((K,), dtype=jnp.float32)
    return weights, moving_avg, relevances, neighbors, wins


# speedup vs baseline: 1.0409x; 1.0409x over previous
import jax
import jax.numpy as jnp
from jax import lax
from jax.experimental import pallas as pl
from jax.experimental.pallas import tpu as pltpu
from jax.experimental.pallas import tpu_sc as plsc

N = 100000
DIM = 128
K = 64
NC = 2
NS = 16
L = 16

RPT = N // NS
CHUNK = 250
NCHUNKS = RPT // CHUNK
NBUF = 6
DEFER = 2
GRP = 125
GPC = CHUNK // GRP
YROWS = N // GRP
YPT = RPT // GRP
KPT = K // NS
CW = DIM // NC


def _seg_mean_body(x_hbm, y_hbm, out_hbm, xb0, xb1, xb2, xb3, xb4, xb5,
                   ybuf, hist, wbuf, cbuf, acc, cstage, sf0, sf1, sf2,
                   sf3, sf4, sf5, ss0, ss1, ss2, ss3, ss4, ss5):
    c = lax.axis_index("c")
    s = lax.axis_index("s")

    zero16 = jnp.zeros((L,), jnp.float32)
    one16 = jnp.ones((L,), jnp.float32)
    for r in range(KPT):
        for j in range(CW // L):
            wbuf[r, pl.ds(L * j, L)] = zero16
    for kk in range(K // L):
        hist[pl.ds(L * kk, L)] = zero16

    pltpu.sync_copy(wbuf, acc.at[pl.ds(s * KPT, KPT)])
    pltpu.sync_copy(y_hbm.at[pl.ds(s * YPT, YPT)], ybuf)
    plsc.subcore_barrier()

    xb = (xb0, xb1, xb2, xb3, xb4, xb5)
    sf = (sf0, sf1, sf2, sf3, sf4, sf5)
    ss = (ss0, ss1, ss2, ss3, ss4, ss5)

    def fill(j):
        row0 = s * RPT + j * CHUNK
        return pltpu.async_copy(
            x_hbm.at[pl.ds(row0, CHUNK), pl.ds(c * CW, CW)],
            xb[j % NBUF], sf[j % NBUF])

    fills = {}
    for j in range(NBUF):
        fills[j] = fill(j)

    tail_mask = lax.iota(jnp.int32, L) >= (L - GRP % L)
    for r in range(YPT):
        for i in range(GRP // L):
            lv = ybuf[r, pl.ds(L * i, L)]
            plsc.addupdate_scatter(hist, [lv], one16)
        lv = ybuf[r, pl.ds(GRP - L, L)]
        plsc.addupdate_scatter(hist, [lv], one16, mask=tail_mask)

    scats = {}
    for j in range(NCHUNKS):
        b = j % NBUF
        fills[j].wait()
        scats[j] = []
        for g in range(GPC):
            idx = ybuf.at[j * GPC + g]
            scats[j].append(
                pltpu.async_copy(xb[b].at[pl.ds(g * GRP, GRP)],
                                 acc.at[idx], ss[b], add=True))
        jd = j - DEFER
        if jd >= 0:
            for h in scats.pop(jd):
                h.wait()
            nf = jd + NBUF
            if nf < NCHUNKS:
                fills[nf] = fill(nf)
    for jd in range(NCHUNKS - DEFER, NCHUNKS):
        for h in scats.pop(jd):
            h.wait()

    pltpu.sync_copy(hist, cstage.at[s])
    plsc.subcore_barrier()

    pltpu.sync_copy(acc.at[pl.ds(s * KPT, KPT)], wbuf)
    pltpu.sync_copy(cstage, cbuf)
    rows = lax.iota(jnp.int32, L)
    for r in range(KPT):
        col = jnp.full((L,), s * KPT + r, jnp.int32)
        parts = plsc.load_gather(cbuf, [rows, col])
        total = jnp.sum(parts, axis=0)
        cnt = jnp.maximum(jnp.full((L,), total, jnp.float32), 1.0)
        for j in range(CW // L):
            wbuf[r, pl.ds(L * j, L)] = wbuf[r, pl.ds(L * j, L)] / cnt
    pltpu.sync_copy(wbuf, out_hbm.at[pl.ds(s * KPT, KPT),
                                     pl.ds(c * CW, CW)])


@jax.jit
def _seg_mean(x, y2):
    return pl.kernel(
        _seg_mean_body,
        out_type=jax.ShapeDtypeStruct((K, DIM), jnp.float32),
        mesh=plsc.VectorSubcoreMesh(core_axis_name="c",
                                    subcore_axis_name="s"),
        scratch_types=[
            pltpu.VMEM((CHUNK, CW), jnp.float32),
            pltpu.VMEM((CHUNK, CW), jnp.float32),
            pltpu.VMEM((CHUNK, CW), jnp.float32),
            pltpu.VMEM((CHUNK, CW), jnp.float32),
            pltpu.VMEM((CHUNK, CW), jnp.float32),
            pltpu.VMEM((CHUNK, CW), jnp.float32),
            pltpu.VMEM((YPT, GRP), jnp.int32),
            pltpu.VMEM((K,), jnp.float32),
            pltpu.VMEM((KPT, CW), jnp.float32),
            pltpu.VMEM((NS, K), jnp.float32),
            pltpu.VMEM_SHARED((K, CW), jnp.float32),
            pltpu.VMEM_SHARED((NS, K), jnp.float32),
            pltpu.SemaphoreType.DMA,
            pltpu.SemaphoreType.DMA,
            pltpu.SemaphoreType.DMA,
            pltpu.SemaphoreType.DMA,
            pltpu.SemaphoreType.DMA,
            pltpu.SemaphoreType.DMA,
            pltpu.SemaphoreType.DMA,
            pltpu.SemaphoreType.DMA,
            pltpu.SemaphoreType.DMA,
            pltpu.SemaphoreType.DMA,
            pltpu.SemaphoreType.DMA,
            pltpu.SemaphoreType.DMA,
        ],
        compiler_params=pltpu.CompilerParams(use_tc_tiling_on_sc=False,
                                             needs_layout_passes=False),
    )(x, y2)


def kernel(x, y):
    y2 = y.astype(jnp.int32).reshape(YROWS, GRP)
    weights = _seg_mean(x, y2)
    moving_avg = jnp.zeros((K, DIM), dtype=jnp.float32)
    relevances = jnp.ones((K, DIM), dtype=jnp.float32)
    neighbors = jnp.zeros((K, K), dtype=jnp.uint8)
    wins = jnp.zeros((K,), dtype=jnp.float32)
    return weights, moving_avg, relevances, neighbors, wins
